# baseline (speedup 1.0000x reference)
import functools

import jax
import jax.numpy as jnp
from jax import lax
from jax.experimental import pallas as pl
from jax.experimental.pallas import tpu as pltpu
from jax.experimental.pallas import tpu_sc as plsc

NC = 2
NS = 16
NW = NC * NS
CHUNK = 128
NBUF = 5


def _make_gather(V, D, B):
    assert B % (NW * CHUNK) == 0
    bpw = B // NW
    nch = bpw // CHUNK
    assert nch % NBUF == 0
    mesh = plsc.VectorSubcoreMesh(
        core_axis_name="c", subcore_axis_name="s",
        num_cores=NC, num_subcores=NS)

    @functools.partial(
        pl.kernel,
        out_type=jax.ShapeDtypeStruct((B, D), jnp.float32),
        mesh=mesh,
        scratch_types=[
            pltpu.VMEM((nch, CHUNK), jnp.int32),
            [pltpu.VMEM((CHUNK, D), jnp.float32)] * NBUF,
            [pltpu.SemaphoreType.DMA] * NBUF,
            [pltpu.SemaphoreType.DMA] * NBUF,
        ],
    )
    def gather_kernel(table_hbm, idx_hbm, out_hbm, idx_v, bufs, gsems, ssems):
        wid = lax.axis_index("s") * NC + lax.axis_index("c")
        base = wid * bpw
        pltpu.sync_copy(idx_hbm.at[wid], idx_v)

        def out_slice(j):
            return out_hbm.at[pl.ds(base + j * CHUNK, CHUNK)]

        for b in range(NBUF):
            pltpu.async_copy(table_hbm.at[idx_v.at[b]], bufs[b], gsems[b])

        @pl.loop(0, nch // NBUF)
        def _(g):
            j0 = g * NBUF
            for b in range(NBUF):
                pltpu.make_async_copy(
                    table_hbm.at[idx_v.at[j0 + b]], bufs[b], gsems[b]).wait()
                pltpu.async_copy(bufs[b], out_slice(j0 + b), ssems[b])
            for b in range(NBUF):
                @pl.when(j0 + b + NBUF < nch)
                def _():
                    pltpu.make_async_copy(
                        bufs[b], out_slice(j0 + b), ssems[b]).wait()
                    pltpu.async_copy(
                        table_hbm.at[idx_v.at[j0 + b + NBUF]],
                        bufs[b], gsems[b])

        for b in range(NBUF):
            pltpu.make_async_copy(
                bufs[b], out_slice(nch - NBUF + b), ssems[b]).wait()

    return gather_kernel


def kernel(x, table):
    V, D = table.shape
    B = x.size
    idx = x.reshape(NW, B // (NW * CHUNK), CHUNK).astype(jnp.int32)
    out = _make_gather(V, D, B)(table, idx)
    return out.reshape(x.shape + (D,))


# speedup vs baseline: 1.7442x; 1.7442x over previous
import functools

import jax
import jax.numpy as jnp
from jax import lax
from jax.experimental import pallas as pl
from jax.experimental.pallas import tpu as pltpu
from jax.experimental.pallas import tpu_sc as plsc

NC = 2
NS = 16
NW = NC * NS
NBUF = 4


def _make_gather(V, D, S, T):
    assert S % (NW * NBUF) == 0
    spw = S // NW
    mesh = plsc.VectorSubcoreMesh(
        core_axis_name="c", subcore_axis_name="s",
        num_cores=NC, num_subcores=NS)

    @functools.partial(
        pl.kernel,
        out_type=jax.ShapeDtypeStruct((S, T, D), jnp.float32),
        mesh=mesh,
        compiler_params=pltpu.CompilerParams(use_tc_tiling_on_sc=True),
        scratch_types=[
            pltpu.VMEM((spw, T), jnp.int32),
            [pltpu.VMEM((T, D), jnp.float32)] * NBUF,
            [pltpu.SemaphoreType.DMA] * NBUF,
            [pltpu.SemaphoreType.DMA] * NBUF,
        ],
    )
    def gather_kernel(table_hbm, idx_hbm, out_hbm, idx_v, bufs, gsems, ssems):
        wid = lax.axis_index("s") * NC + lax.axis_index("c")
        base = wid * spw
        pltpu.sync_copy(idx_hbm.at[wid], idx_v)

        for b in range(NBUF):
            pltpu.async_copy(table_hbm.at[idx_v.at[b]], bufs[b], gsems[b])

        @pl.loop(0, spw // NBUF)
        def _(g):
            t0 = g * NBUF
            for b in range(NBUF):
                pltpu.make_async_copy(
                    table_hbm.at[idx_v.at[t0 + b]], bufs[b], gsems[b]).wait()
                pltpu.async_copy(bufs[b], out_hbm.at[base + t0 + b], ssems[b])
            for b in range(NBUF):
                @pl.when(t0 + b + NBUF < spw)
                def _():
                    pltpu.make_async_copy(
                        bufs[b], out_hbm.at[base + t0 + b], ssems[b]).wait()
                    pltpu.async_copy(
                        table_hbm.at[idx_v.at[t0 + b + NBUF]],
                        bufs[b], gsems[b])

        for b in range(NBUF):
            pltpu.make_async_copy(
                bufs[b], out_hbm.at[base + spw - NBUF + b], ssems[b]).wait()

    return gather_kernel


def kernel(x, table):
    V, D = table.shape
    S, T = x.shape
    idx = x.reshape(NW, S // NW, T).astype(jnp.int32)
    return _make_gather(V, D, S, T)(table, idx)


# speedup vs baseline: 3.0470x; 1.7469x over previous
import functools

import jax
import jax.numpy as jnp
from jax import lax
from jax.experimental import pallas as pl
from jax.experimental.pallas import tpu as pltpu
from jax.experimental.pallas import tpu_sc as plsc

NC = 2
NS = 16
NW = NC * NS
CHUNK = 128
NBUF = 5


def _make_gather(V, D, B):
    assert B % (NW * CHUNK) == 0
    bpw = B // NW
    nch = bpw // CHUNK
    assert nch % NBUF == 0
    mesh = plsc.VectorSubcoreMesh(
        core_axis_name="c", subcore_axis_name="s",
        num_cores=NC, num_subcores=NS)

    @functools.partial(
        pl.kernel,
        out_type=jax.ShapeDtypeStruct((B, D), jnp.float32),
        mesh=mesh,
        scratch_types=[
            pltpu.VMEM((nch, CHUNK), jnp.int32),
            [pltpu.VMEM((CHUNK, D), jnp.float32)] * NBUF,
            [pltpu.SemaphoreType.DMA] * NBUF,
            [pltpu.SemaphoreType.DMA] * NBUF,
        ],
    )
    def gather_kernel(table_hbm, idx_hbm, out_hbm, idx_v, bufs, gsems, ssems):
        wid = lax.axis_index("s") * NC + lax.axis_index("c")
        base = wid * bpw
        pltpu.sync_copy(idx_hbm.at[wid], idx_v)

        def out_slice(j):
            return out_hbm.at[pl.ds(base + j * CHUNK, CHUNK)]

        for b in range(NBUF):
            pltpu.async_copy(table_hbm.at[idx_v.at[b]], bufs[b], gsems[b])

        @pl.loop(0, nch // NBUF)
        def _(g):
            j0 = g * NBUF
            for b in range(NBUF):
                pltpu.make_async_copy(
                    table_hbm.at[idx_v.at[j0 + b]], bufs[b], gsems[b]).wait()
                pltpu.async_copy(bufs[b], out_slice(j0 + b), ssems[b])
            for b in range(NBUF):
                @pl.when(j0 + b + NBUF < nch)
                def _():
                    pltpu.make_async_copy(
                        bufs[b], out_slice(j0 + b), ssems[b]).wait()
                    pltpu.async_copy(
                        table_hbm.at[idx_v.at[j0 + b + NBUF]],
                        bufs[b], gsems[b])

        for b in range(NBUF):
            pltpu.make_async_copy(
                bufs[b], out_slice(nch - NBUF + b), ssems[b]).wait()

    return gather_kernel


def kernel(x, table):
    V, D = table.shape
    S, T = x.shape
    B = x.size
    idx = x.T.reshape(NW, B // (NW * CHUNK), CHUNK).astype(jnp.int32)
    out = _make_gather(V, D, B)(table, idx)
    return out.reshape(T, S, D).transpose(1, 0, 2)
